# Initial kernel scaffold; baseline (speedup 1.0000x reference)
import jax
import jax.numpy as jnp
from jax.experimental import pallas as pl


def kernel(features, edge_index, W1, b1, W2, b2, W3, b3, Wi, bi):
    raise NotImplementedError("write your pallas kernel here")


# speedup vs baseline: 4.1839x; 4.1839x over previous
import functools

import jax
import jax.numpy as jnp
from jax import lax
from jax.experimental import pallas as pl
from jax.experimental.pallas import tpu as pltpu
from jax.experimental.pallas import tpu_sc as plsc

N = 10000
E = 160000
NSUB = 16
BLK = 128
NBLK = 80
E_PAD = NSUB * NBLK * BLK
NACC = 10240
STRIPE = NACC // NSUB
CW = 128

NC1 = 6
NC2 = 4

MM_BN = 1000
MM_GRID = N // MM_BN


def _mesh():
    return plsc.VectorSubcoreMesh(core_axis_name="c", subcore_axis_name="s")


def _segsum_scratch():
    return [
        pltpu.VMEM((NBLK, BLK), jnp.int32),
        pltpu.VMEM((NBLK, BLK), jnp.int32),
        pltpu.VMEM((BLK, CW), jnp.float32),
        pltpu.VMEM_SHARED((NACC, CW), jnp.float32),
        pltpu.SemaphoreType.DMA,
    ]


def _edge_loop(z_ref, sidx, didx, rows, acc, sem, lo, hi):
    def step(j, carry):
        pltpu.async_copy(z_ref.at[sidx.at[j]], rows, sem).wait()
        pltpu.sync_copy(rows, acc.at[didx.at[j]], add=True)
        return carry

    lax.fori_loop(lo, hi, step, 0)


def _make_segsum_cols(nchunks):
    out_type = tuple(jax.ShapeDtypeStruct((NACC, CW), jnp.float32)
                     for _ in range(nchunks))

    def body(*refs):
        z_refs = refs[:nchunks]
        src_ref, dst_ref, zero_ref = refs[nchunks:nchunks + 3]
        out_refs = refs[nchunks + 3:2 * nchunks + 3]
        sidx, didx, rows, acc, sem = refs[2 * nchunks + 3:]
        core = lax.axis_index("c")
        t = lax.axis_index("s")

        pltpu.sync_copy(src_ref.at[t], sidx)
        pltpu.sync_copy(dst_ref.at[t], didx)

        for c in range(nchunks):
            @pl.when(core == (c % 2))
            def _(c=c):
                pltpu.sync_copy(zero_ref, acc.at[pl.ds(t * STRIPE, STRIPE)])
                plsc.subcore_barrier()
                _edge_loop(z_refs[c], sidx, didx, rows, acc, sem, 0, NBLK)
                plsc.subcore_barrier()
                pltpu.sync_copy(acc.at[pl.ds(t * STRIPE, STRIPE)],
                                out_refs[c].at[pl.ds(t * STRIPE, STRIPE)])
                plsc.subcore_barrier()

    return pl.kernel(body, out_type=out_type, mesh=_mesh(),
                     scratch_types=_segsum_scratch())


def _make_segsum_edges():
    out_type = (jax.ShapeDtypeStruct((NACC, CW), jnp.float32),
                jax.ShapeDtypeStruct((NACC, CW), jnp.float32))
    half = NBLK // 2

    def body(z_ref, src_ref, dst_ref, zero_ref, out0, out1,
             sidx, didx, rows, acc, sem):
        core = lax.axis_index("c")
        t = lax.axis_index("s")

        pltpu.sync_copy(src_ref.at[t], sidx)
        pltpu.sync_copy(dst_ref.at[t], didx)
        pltpu.sync_copy(zero_ref, acc.at[pl.ds(t * STRIPE, STRIPE)])
        plsc.subcore_barrier()
        for k, out_ref in enumerate((out0, out1)):
            @pl.when(core == k)
            def _(k=k, out_ref=out_ref):
                _edge_loop(z_ref, sidx, didx, rows, acc, sem,
                           k * half, (k + 1) * half)
                plsc.subcore_barrier()
                pltpu.sync_copy(acc.at[pl.ds(t * STRIPE, STRIPE)],
                                out_ref.at[pl.ds(t * STRIPE, STRIPE)])

    return pl.kernel(body, out_type=out_type, mesh=_mesh(),
                     scratch_types=_segsum_scratch())


def _mm1_body(x_ref, w_ref, *out_refs):
    z = jnp.dot(x_ref[...], w_ref[...], preferred_element_type=jnp.float32)
    for c in range(NC1):
        out_refs[c][...] = z[:, c * CW:(c + 1) * CW]


def _mm2_body(*refs):
    a_refs = refs[:NC1]
    b_ref, w_ref = refs[NC1:NC1 + 2]
    out_refs = refs[NC1 + 2:]
    acc = None
    for c in range(NC1):
        xc = jnp.maximum(a_refs[c][...] + b_ref[0, c * CW:(c + 1) * CW], 0.0)
        p = jnp.dot(xc, w_ref[c * CW:(c + 1) * CW, :],
                    preferred_element_type=jnp.float32)
        acc = p if acc is None else acc + p
    for c in range(NC2):
        out_refs[c][...] = acc[:, c * CW:(c + 1) * CW]


def _mm3_body(a0, a1, a2, a3, b_ref, w3_ref, wi_ref, out_ref):
    a_refs = (a0, a1, a2, a3)
    acc = None
    for c in range(NC2):
        xc = jnp.maximum(a_refs[c][...] + b_ref[0, c * CW:(c + 1) * CW], 0.0)
        p = jnp.dot(xc, w3_ref[c * CW:(c + 1) * CW, :],
                    preferred_element_type=jnp.float32)
        acc = p if acc is None else acc + p
    out_ref[...] = jnp.dot(acc, wi_ref[...], preferred_element_type=jnp.float32)


def _final_body(p0, p1, b3_ref, wi_ref, bi_ref, out_ref):
    bv = jnp.dot(b3_ref[...], wi_ref[...],
                 preferred_element_type=jnp.float32) + bi_ref[...]
    h = p0[:, :64] + p1[:, :64] + bv
    out_ref[...] = jnp.maximum(h, 0.0)


def _row_spec(shape):
    return pl.BlockSpec((MM_BN,) + shape[1:], lambda i: (i,) + (0,) * (len(shape) - 1))


def _full_spec(shape):
    return pl.BlockSpec(shape, lambda i: (0,) * len(shape))


def kernel(features, edge_index, W1, b1, W2, b2, W3, b3, Wi, bi):
    f32 = jnp.float32
    W1p = jnp.pad(W1, ((0, 0), (0, NC1 * CW - 700)))
    b1p = jnp.pad(b1, (0, NC1 * CW - 700)).reshape(1, NC1 * CW)
    W2p = jnp.pad(W2, ((0, NC1 * CW - 700), (0, NC2 * CW - 400)))
    b2p = jnp.pad(b2, (0, NC2 * CW - 400)).reshape(1, NC2 * CW)
    W3p = jnp.pad(W3, ((0, NC2 * CW - 400), (0, 0)))
    Wip = jnp.pad(Wi, ((0, 0), (0, CW - 64)))
    b3r = b3.reshape(1, 100)
    bir = bi.reshape(1, 64)

    src = edge_index[0]
    dst = edge_index[1]
    pad = E_PAD - E
    srcp = jnp.concatenate([src, jnp.zeros((pad,), jnp.int32)]).reshape(NSUB, NBLK, BLK)
    dstp = jnp.concatenate([dst, jnp.full((pad,), N, jnp.int32)]).reshape(NSUB, NBLK, BLK)
    zeros = jnp.zeros((STRIPE, CW), f32)

    mm1 = pl.pallas_call(
        _mm1_body,
        grid=(MM_GRID,),
        in_specs=[_row_spec((N, 1433)), _full_spec((1433, NC1 * CW))],
        out_specs=[_row_spec((N, CW))] * NC1,
        out_shape=[jax.ShapeDtypeStruct((N, CW), f32)] * NC1,
    )
    z1 = mm1(features, W1p)

    a1 = _make_segsum_cols(NC1)(*z1, srcp, dstp, zeros)

    mm2 = pl.pallas_call(
        _mm2_body,
        grid=(MM_GRID,),
        in_specs=[_row_spec((NACC, CW))] * NC1 + [
            _full_spec((1, NC1 * CW)), _full_spec((NC1 * CW, NC2 * CW))],
        out_specs=[_row_spec((N, CW))] * NC2,
        out_shape=[jax.ShapeDtypeStruct((N, CW), f32)] * NC2,
    )
    z2 = mm2(*a1, b1p, W2p)

    a2 = _make_segsum_cols(NC2)(*z2, srcp, dstp, zeros)

    mm3 = pl.pallas_call(
        _mm3_body,
        grid=(MM_GRID,),
        in_specs=[_row_spec((NACC, CW))] * NC2 + [
            _full_spec((1, NC2 * CW)), _full_spec((NC2 * CW, 100)),
            _full_spec((100, CW))],
        out_specs=_row_spec((N, CW)),
        out_shape=jax.ShapeDtypeStruct((N, CW), f32),
    )
    z3 = mm3(*a2, b2p, W3p, Wip)

    a3 = _make_segsum_edges()(z3, srcp, dstp, zeros)

    fin = pl.pallas_call(
        _final_body,
        grid=(MM_GRID,),
        in_specs=[_row_spec((NACC, CW))] * 2 + [
            _full_spec((1, 100)), _full_spec((100, 64)), _full_spec((1, 64))],
        out_specs=_row_spec((N, 64)),
        out_shape=jax.ShapeDtypeStruct((N, 64), f32),
    )
    return fin(*a3, b3r, Wi, bir)


# speedup vs baseline: 5.1800x; 1.2381x over previous
import functools

import jax
import jax.numpy as jnp
from jax import lax
from jax.experimental import pallas as pl
from jax.experimental.pallas import tpu as pltpu
from jax.experimental.pallas import tpu_sc as plsc

N = 10000
E = 160000
NSUB = 16
BLK = 128
NBLK = 80
E_PAD = NSUB * NBLK * BLK
NACC = 10112
STRIPE = NACC // NSUB
CW = 128
WIN = 40
NWIN = NBLK // WIN

NC1 = 6
NC2 = 4

MM_BN = 1000
MM_GRID = N // MM_BN


def _mesh():
    return plsc.VectorSubcoreMesh(core_axis_name="c", subcore_axis_name="s")


def _segsum_scratch():
    return [
        pltpu.VMEM((WIN, BLK), jnp.int32),
        pltpu.VMEM((WIN, BLK), jnp.int32),
        pltpu.VMEM((2, BLK, CW), jnp.float32),
        pltpu.VMEM_SHARED((NACC, CW), jnp.float32),
        pltpu.SemaphoreType.DMA,
        pltpu.SemaphoreType.DMA,
    ]


def _edge_window(z_ref, sidx, didx, rows, acc, sems):
    npairs = WIN // 2

    def gather(j, b):
        return pltpu.async_copy(z_ref.at[sidx.at[j]], rows.at[b], sems[b])

    def wait(b):
        pltpu.make_async_copy(z_ref.at[sidx.at[0]], rows.at[b], sems[b]).wait()

    gather(0, 0)
    gather(1, 1)

    def step(m, carry):
        j = 2 * m
        for b in range(2):
            wait(b)
            pltpu.sync_copy(rows.at[b], acc.at[didx.at[j + b]], add=True)

            @pl.when(m < npairs - 1)
            def _(b=b):
                gather(j + 2 + b, b)
        return carry

    lax.fori_loop(0, npairs, step, 0)


def _edge_loop(z_ref, src_ref, dst_ref, t, sidx, didx, rows, acc, sems, windows):
    for w in windows:
        pltpu.sync_copy(src_ref.at[t, pl.ds(w * WIN, WIN)], sidx)
        pltpu.sync_copy(dst_ref.at[t, pl.ds(w * WIN, WIN)], didx)
        _edge_window(z_ref, sidx, didx, rows, acc, sems)


def _make_segsum_cols(nchunks):
    out_type = tuple(jax.ShapeDtypeStruct((NACC, CW), jnp.float32)
                     for _ in range(nchunks))

    def body(*refs):
        z_refs = refs[:nchunks]
        src_ref, dst_ref, zero_ref = refs[nchunks:nchunks + 3]
        out_refs = refs[nchunks + 3:2 * nchunks + 3]
        sidx, didx, rows, acc, semA, semB = refs[2 * nchunks + 3:]
        core = lax.axis_index("c")
        t = lax.axis_index("s")

        for c in range(nchunks):
            @pl.when(core == (c % 2))
            def _(c=c):
                pltpu.sync_copy(zero_ref, acc.at[pl.ds(t * STRIPE, STRIPE)])
                plsc.subcore_barrier()
                _edge_loop(z_refs[c], src_ref, dst_ref, t, sidx, didx, rows,
                           acc, (semA, semB), range(NWIN))
                plsc.subcore_barrier()
                pltpu.sync_copy(acc.at[pl.ds(t * STRIPE, STRIPE)],
                                out_refs[c].at[pl.ds(t * STRIPE, STRIPE)])
                plsc.subcore_barrier()

    return pl.kernel(body, out_type=out_type, mesh=_mesh(),
                     scratch_types=_segsum_scratch())


def _make_segsum_edges():
    out_type = (jax.ShapeDtypeStruct((NACC, CW), jnp.float32),
                jax.ShapeDtypeStruct((NACC, CW), jnp.float32))

    def body(z_ref, src_ref, dst_ref, zero_ref, out0, out1,
             sidx, didx, rows, acc, semA, semB):
        core = lax.axis_index("c")
        t = lax.axis_index("s")

        pltpu.sync_copy(zero_ref, acc.at[pl.ds(t * STRIPE, STRIPE)])
        plsc.subcore_barrier()
        for k, out_ref in enumerate((out0, out1)):
            @pl.when(core == k)
            def _(k=k, out_ref=out_ref):
                _edge_loop(z_ref, src_ref, dst_ref, t, sidx, didx, rows,
                           acc, (semA, semB), [k])
                plsc.subcore_barrier()
                pltpu.sync_copy(acc.at[pl.ds(t * STRIPE, STRIPE)],
                                out_ref.at[pl.ds(t * STRIPE, STRIPE)])

    return pl.kernel(body, out_type=out_type, mesh=_mesh(),
                     scratch_types=_segsum_scratch())


def _mm1_body(x_ref, w_ref, *out_refs):
    z = jnp.dot(x_ref[...], w_ref[...], preferred_element_type=jnp.float32)
    for c in range(NC1):
        out_refs[c][...] = z[:, c * CW:(c + 1) * CW]


def _mm2_body(*refs):
    a_refs = refs[:NC1]
    b_ref, w_ref = refs[NC1:NC1 + 2]
    out_refs = refs[NC1 + 2:]
    acc = None
    for c in range(NC1):
        xc = jnp.maximum(a_refs[c][...] + b_ref[0, c * CW:(c + 1) * CW], 0.0)
        p = jnp.dot(xc, w_ref[c * CW:(c + 1) * CW, :],
                    preferred_element_type=jnp.float32)
        acc = p if acc is None else acc + p
    for c in range(NC2):
        out_refs[c][...] = acc[:, c * CW:(c + 1) * CW]


def _mm3_body(a0, a1, a2, a3, b_ref, w3_ref, wi_ref, out_ref):
    a_refs = (a0, a1, a2, a3)
    acc = None
    for c in range(NC2):
        xc = jnp.maximum(a_refs[c][...] + b_ref[0, c * CW:(c + 1) * CW], 0.0)
        p = jnp.dot(xc, w3_ref[c * CW:(c + 1) * CW, :],
                    preferred_element_type=jnp.float32)
        acc = p if acc is None else acc + p
    out_ref[...] = jnp.dot(acc, wi_ref[...], preferred_element_type=jnp.float32)


def _final_body(p0, p1, b3_ref, wi_ref, bi_ref, out_ref):
    bv = jnp.dot(b3_ref[...], wi_ref[...],
                 preferred_element_type=jnp.float32) + bi_ref[...]
    h = p0[:, :64] + p1[:, :64] + bv
    out_ref[...] = jnp.maximum(h, 0.0)


def _row_spec(shape):
    return pl.BlockSpec((MM_BN,) + shape[1:], lambda i: (i,) + (0,) * (len(shape) - 1))


def _full_spec(shape):
    return pl.BlockSpec(shape, lambda i: (0,) * len(shape))


def kernel(features, edge_index, W1, b1, W2, b2, W3, b3, Wi, bi):
    f32 = jnp.float32
    W1p = jnp.pad(W1, ((0, 0), (0, NC1 * CW - 700)))
    b1p = jnp.pad(b1, (0, NC1 * CW - 700)).reshape(1, NC1 * CW)
    W2p = jnp.pad(W2, ((0, NC1 * CW - 700), (0, NC2 * CW - 400)))
    b2p = jnp.pad(b2, (0, NC2 * CW - 400)).reshape(1, NC2 * CW)
    W3p = jnp.pad(W3, ((0, NC2 * CW - 400), (0, 0)))
    Wip = jnp.pad(Wi, ((0, 0), (0, CW - 64)))
    b3r = b3.reshape(1, 100)
    bir = bi.reshape(1, 64)

    src = edge_index[0]
    dst = edge_index[1]
    pad = E_PAD - E
    srcp = jnp.concatenate([src, jnp.zeros((pad,), jnp.int32)]).reshape(NSUB, NBLK, BLK)
    dstp = jnp.concatenate([dst, jnp.full((pad,), N, jnp.int32)]).reshape(NSUB, NBLK, BLK)
    zeros = jnp.zeros((STRIPE, CW), f32)

    mm1 = pl.pallas_call(
        _mm1_body,
        grid=(MM_GRID,),
        in_specs=[_row_spec((N, 1433)), _full_spec((1433, NC1 * CW))],
        out_specs=[_row_spec((N, CW))] * NC1,
        out_shape=[jax.ShapeDtypeStruct((N, CW), f32)] * NC1,
    )
    z1 = mm1(features, W1p)

    a1 = _make_segsum_cols(NC1)(*z1, srcp, dstp, zeros)

    mm2 = pl.pallas_call(
        _mm2_body,
        grid=(MM_GRID,),
        in_specs=[_row_spec((NACC, CW))] * NC1 + [
            _full_spec((1, NC1 * CW)), _full_spec((NC1 * CW, NC2 * CW))],
        out_specs=[_row_spec((N, CW))] * NC2,
        out_shape=[jax.ShapeDtypeStruct((N, CW), f32)] * NC2,
    )
    z2 = mm2(*a1, b1p, W2p)

    a2 = _make_segsum_cols(NC2)(*z2, srcp, dstp, zeros)

    mm3 = pl.pallas_call(
        _mm3_body,
        grid=(MM_GRID,),
        in_specs=[_row_spec((NACC, CW))] * NC2 + [
            _full_spec((1, NC2 * CW)), _full_spec((NC2 * CW, 100)),
            _full_spec((100, CW))],
        out_specs=_row_spec((N, CW)),
        out_shape=jax.ShapeDtypeStruct((N, CW), f32),
    )
    z3 = mm3(*a2, b2p, W3p, Wip)

    a3 = _make_segsum_edges()(z3, srcp, dstp, zeros)

    fin = pl.pallas_call(
        _final_body,
        grid=(MM_GRID,),
        in_specs=[_row_spec((NACC, CW))] * 2 + [
            _full_spec((1, 100)), _full_spec((100, 64)), _full_spec((1, 64))],
        out_specs=_row_spec((N, 64)),
        out_shape=jax.ShapeDtypeStruct((N, 64), f32),
    )
    return fin(*a3, b3r, Wi, bir)


# speedup vs baseline: 5.1816x; 1.0003x over previous
import functools

import jax
import jax.numpy as jnp
from jax import lax
from jax.experimental import pallas as pl
from jax.experimental.pallas import tpu as pltpu
from jax.experimental.pallas import tpu_sc as plsc

N = 10000
E = 160000
NSUB = 16
BLK = 128
NBLK = 80
E_PAD = NSUB * NBLK * BLK
NACC = 10112
STRIPE = NACC // NSUB
CW = 128
WIN = 40
NWIN = NBLK // WIN

NC1 = 6
NC2 = 4

MM_BN = 1000
MM_GRID = N // MM_BN


def _mesh():
    return plsc.VectorSubcoreMesh(core_axis_name="c", subcore_axis_name="s")


def _segsum_scratch():
    return [
        pltpu.VMEM((WIN, BLK), jnp.int32),
        pltpu.VMEM((WIN, BLK), jnp.int32),
        pltpu.VMEM((2, BLK, CW), jnp.float32),
        pltpu.VMEM_SHARED((NACC, CW), jnp.float32),
        pltpu.SemaphoreType.DMA,
        pltpu.SemaphoreType.DMA,
    ]


def _edge_window(z_ref, sidx, didx, rows, acc, sems):
    npairs = WIN // 2

    def gather(j, b):
        return pltpu.async_copy(z_ref.at[sidx.at[j]], rows.at[b], sems[b])

    def wait(b):
        pltpu.make_async_copy(z_ref.at[sidx.at[0]], rows.at[b], sems[b]).wait()

    gather(0, 0)
    gather(1, 1)

    def step(m, carry):
        j = 2 * m
        for b in range(2):
            wait(b)
            pltpu.sync_copy(rows.at[b], acc.at[didx.at[j + b]], add=True)

            @pl.when(m < npairs - 1)
            def _(b=b):
                gather(j + 2 + b, b)
        return carry

    lax.fori_loop(0, npairs, step, 0)


def _edge_loop(z_ref, src_ref, dst_ref, t, sidx, didx, rows, acc, sems, windows):
    for w in windows:
        pltpu.sync_copy(src_ref.at[t, pl.ds(w * WIN, WIN)], sidx)
        pltpu.sync_copy(dst_ref.at[t, pl.ds(w * WIN, WIN)], didx)
        _edge_window(z_ref, sidx, didx, rows, acc, sems)


def _make_segsum_cols(nchunks):
    out_type = tuple(jax.ShapeDtypeStruct((NACC, CW), jnp.float32)
                     for _ in range(nchunks))

    def body(*refs):
        z_refs = refs[:nchunks]
        src_ref, dst_ref, zero_ref = refs[nchunks:nchunks + 3]
        out_refs = refs[nchunks + 3:2 * nchunks + 3]
        sidx, didx, rows, acc, semA, semB = refs[2 * nchunks + 3:]
        core = lax.axis_index("c")
        t = lax.axis_index("s")

        for c in range(nchunks):
            @pl.when(core == (c % 2))
            def _(c=c):
                pltpu.sync_copy(zero_ref, acc.at[pl.ds(t * STRIPE, STRIPE)])
                plsc.subcore_barrier()
                _edge_loop(z_refs[c], src_ref, dst_ref, t, sidx, didx, rows,
                           acc, (semA, semB), range(NWIN))
                plsc.subcore_barrier()
                pltpu.sync_copy(acc.at[pl.ds(t * STRIPE, STRIPE)],
                                out_refs[c].at[pl.ds(t * STRIPE, STRIPE)])
                plsc.subcore_barrier()

    return pl.kernel(body, out_type=out_type, mesh=_mesh(),
                     scratch_types=_segsum_scratch())


def _make_segsum_edges():
    out_type = (jax.ShapeDtypeStruct((NACC, CW), jnp.float32),
                jax.ShapeDtypeStruct((NACC, CW), jnp.float32))

    def body(z_ref, src_ref, dst_ref, zero_ref, out0, out1,
             sidx, didx, rows, acc, semA, semB):
        core = lax.axis_index("c")
        t = lax.axis_index("s")

        pltpu.sync_copy(zero_ref, acc.at[pl.ds(t * STRIPE, STRIPE)])
        plsc.subcore_barrier()
        for k, out_ref in enumerate((out0, out1)):
            @pl.when(core == k)
            def _(k=k, out_ref=out_ref):
                _edge_loop(z_ref, src_ref, dst_ref, t, sidx, didx, rows,
                           acc, (semA, semB), [k])
                plsc.subcore_barrier()
                pltpu.sync_copy(acc.at[pl.ds(t * STRIPE, STRIPE)],
                                out_ref.at[pl.ds(t * STRIPE, STRIPE)])

    return pl.kernel(body, out_type=out_type, mesh=_mesh(),
                     scratch_types=_segsum_scratch())


def _mm1_body(x_ref, w_ref, *out_refs):
    z = jnp.dot(x_ref[...], w_ref[...], preferred_element_type=jnp.float32)
    for c in range(NC1):
        out_refs[c][...] = z[:, c * CW:(c + 1) * CW]


def _mm2_body(*refs):
    a_refs = refs[:NC1]
    b_ref, w_ref = refs[NC1:NC1 + 2]
    out_refs = refs[NC1 + 2:]
    acc = None
    for c in range(NC1):
        xc = jnp.maximum(a_refs[c][...] + b_ref[0, c * CW:(c + 1) * CW], 0.0)
        p = jnp.dot(xc, w_ref[c * CW:(c + 1) * CW, :],
                    preferred_element_type=jnp.float32)
        acc = p if acc is None else acc + p
    for c in range(NC2):
        out_refs[c][...] = acc[:, c * CW:(c + 1) * CW]


def _mm3_body(a0, a1, a2, a3, b_ref, w3_ref, wi_ref, out_ref):
    a_refs = (a0, a1, a2, a3)
    acc = None
    for c in range(NC2):
        xc = jnp.maximum(a_refs[c][...] + b_ref[0, c * CW:(c + 1) * CW], 0.0)
        p = jnp.dot(xc, w3_ref[c * CW:(c + 1) * CW, :],
                    preferred_element_type=jnp.float32)
        acc = p if acc is None else acc + p
    out_ref[...] = jnp.dot(acc, wi_ref[...], preferred_element_type=jnp.float32)


def _final_body(p0, p1, b3_ref, wi_ref, bi_ref, out_ref):
    bv = jnp.dot(b3_ref[...], wi_ref[...],
                 preferred_element_type=jnp.float32) + bi_ref[...]
    h = p0[:, :64] + p1[:, :64] + bv
    out_ref[...] = jnp.maximum(h, 0.0)


def _row_spec(shape):
    return pl.BlockSpec((MM_BN,) + shape[1:], lambda i: (i,) + (0,) * (len(shape) - 1))


def _full_spec(shape):
    return pl.BlockSpec(shape, lambda i: (0,) * len(shape))


def kernel(features, edge_index, W1, b1, W2, b2, W3, b3, Wi, bi):
    f32 = jnp.float32
    W1p = jnp.pad(W1, ((0, 0), (0, NC1 * CW - 700)))
    b1p = jnp.pad(b1, (0, NC1 * CW - 700)).reshape(1, NC1 * CW)
    W2p = jnp.pad(W2, ((0, NC1 * CW - 700), (0, NC2 * CW - 400)))
    b2p = jnp.pad(b2, (0, NC2 * CW - 400)).reshape(1, NC2 * CW)
    W3p = jnp.pad(W3, ((0, NC2 * CW - 400), (0, 0)))
    Wip = jnp.pad(Wi, ((0, 0), (0, CW - 64)))
    b3r = b3.reshape(1, 100)
    bir = bi.reshape(1, 64)

    src = edge_index[0]
    dst = edge_index[1]
    pad = E_PAD - E
    srcp = jnp.concatenate([src, jnp.zeros((pad,), jnp.int32)]).reshape(NSUB, NBLK, BLK)
    dump = N + (jnp.arange(pad, dtype=jnp.int32) % (NACC - N))
    dstp = jnp.concatenate([dst, dump]).reshape(NSUB, NBLK, BLK)
    zeros = jnp.zeros((STRIPE, CW), f32)

    mm1 = pl.pallas_call(
        _mm1_body,
        grid=(MM_GRID,),
        in_specs=[_row_spec((N, 1433)), _full_spec((1433, NC1 * CW))],
        out_specs=[_row_spec((N, CW))] * NC1,
        out_shape=[jax.ShapeDtypeStruct((N, CW), f32)] * NC1,
    )
    z1 = mm1(features, W1p)

    a1 = _make_segsum_cols(NC1)(*z1, srcp, dstp, zeros)

    mm2 = pl.pallas_call(
        _mm2_body,
        grid=(MM_GRID,),
        in_specs=[_row_spec((NACC, CW))] * NC1 + [
            _full_spec((1, NC1 * CW)), _full_spec((NC1 * CW, NC2 * CW))],
        out_specs=[_row_spec((N, CW))] * NC2,
        out_shape=[jax.ShapeDtypeStruct((N, CW), f32)] * NC2,
    )
    z2 = mm2(*a1, b1p, W2p)

    a2 = _make_segsum_cols(NC2)(*z2, srcp, dstp, zeros)

    mm3 = pl.pallas_call(
        _mm3_body,
        grid=(MM_GRID,),
        in_specs=[_row_spec((NACC, CW))] * NC2 + [
            _full_spec((1, NC2 * CW)), _full_spec((NC2 * CW, 100)),
            _full_spec((100, CW))],
        out_specs=_row_spec((N, CW)),
        out_shape=jax.ShapeDtypeStruct((N, CW), f32),
    )
    z3 = mm3(*a2, b2p, W3p, Wip)

    a3 = _make_segsum_edges()(z3, srcp, dstp, zeros)

    fin = pl.pallas_call(
        _final_body,
        grid=(MM_GRID,),
        in_specs=[_row_spec((NACC, CW))] * 2 + [
            _full_spec((1, 100)), _full_spec((100, 64)), _full_spec((1, 64))],
        out_specs=_row_spec((N, 64)),
        out_shape=jax.ShapeDtypeStruct((N, 64), f32),
    )
    return fin(*a3, b3r, Wi, bir)
